# Initial kernel scaffold; baseline (speedup 1.0000x reference)
import jax
import jax.numpy as jnp
from jax.experimental import pallas as pl


def kernel(category, weight):
    raise NotImplementedError("write your pallas kernel here")


# speedup vs baseline: 4.0903x; 4.0903x over previous
import functools

import jax
import jax.numpy as jnp
from jax import lax
from jax.experimental import pallas as pl
from jax.experimental.pallas import tpu as pltpu
from jax.experimental.pallas import tpu_sc as plsc

D = 64
B = 4096 * 50

_info = plsc.get_sparse_core_info()
_NC = _info.num_cores
_NS = _info.num_subcores
NW = _NC * _NS
BPW = B // NW
CH = 128
NCH = BPW // CH

_mesh = plsc.VectorSubcoreMesh(core_axis_name="c", subcore_axis_name="s")


@functools.partial(
    pl.kernel,
    mesh=_mesh,
    out_type=jax.ShapeDtypeStruct((NW, NCH, CH, D), jnp.float32),
    scratch_types=[
        pltpu.VMEM((NCH, CH), jnp.int32),
        pltpu.VMEM((CH, D), jnp.float32),
        pltpu.SemaphoreType.DMA,
    ],
    compiler_params=pltpu.CompilerParams(use_tc_tiling_on_sc=False),
)
def _gather_kernel(idx_hbm, table_hbm, out_hbm, idx_v, rows_v, sem):
    wid = lax.axis_index("s") * _NC + lax.axis_index("c")
    pltpu.sync_copy(idx_hbm.at[wid], idx_v)

    def body(j, carry):
        pltpu.async_copy(table_hbm.at[idx_v.at[j]], rows_v, sem).wait()
        pltpu.sync_copy(rows_v, out_hbm.at[wid, j])
        return carry

    lax.fori_loop(0, NCH, body, 0)


def kernel(category, weight):
    idx = category.reshape(NW, NCH, CH)
    out = _gather_kernel(idx, weight)
    return out.reshape(category.shape[0], category.shape[1], D)


# speedup vs baseline: 4.5392x; 1.1097x over previous
import functools

import jax
import jax.numpy as jnp
from jax import lax
from jax.experimental import pallas as pl
from jax.experimental.pallas import tpu as pltpu
from jax.experimental.pallas import tpu_sc as plsc

D = 64
B = 4096 * 50

_info = plsc.get_sparse_core_info()
_NC = _info.num_cores
_NS = _info.num_subcores
NW = _NC * _NS
BPW = B // NW
CH = 128
NCH = BPW // CH

NBUF = 2

_mesh = plsc.VectorSubcoreMesh(core_axis_name="c", subcore_axis_name="s")


@functools.partial(
    pl.kernel,
    mesh=_mesh,
    out_type=jax.ShapeDtypeStruct((NW, NCH, CH, D), jnp.float32),
    scratch_types=[
        pltpu.VMEM((NCH, CH), jnp.int32),
        pltpu.VMEM((NBUF, CH, D), jnp.float32),
        pltpu.SemaphoreType.DMA((NBUF,)),
        pltpu.SemaphoreType.DMA((NBUF,)),
    ],
    compiler_params=pltpu.CompilerParams(use_tc_tiling_on_sc=False),
)
def _gather_kernel(idx_hbm, table_hbm, out_hbm, idx_v, rows_v, gsem, ssem):
    wid = lax.axis_index("s") * _NC + lax.axis_index("c")
    pltpu.sync_copy(idx_hbm.at[wid], idx_v)

    for b in range(NBUF):
        pltpu.async_copy(table_hbm.at[idx_v.at[b]], rows_v.at[b], gsem.at[b])

    def body(k, carry):
        for b in range(NBUF):
            j = k * NBUF + b
            pltpu.make_async_copy(
                table_hbm.at[idx_v.at[b]], rows_v.at[b], gsem.at[b]).wait()
            pltpu.async_copy(rows_v.at[b], out_hbm.at[wid, j], ssem.at[b])

            @pl.when(j + NBUF < NCH)
            def _():
                pltpu.make_async_copy(
                    rows_v.at[b], out_hbm.at[wid, j], ssem.at[b]).wait()
                pltpu.async_copy(
                    table_hbm.at[idx_v.at[j + NBUF]], rows_v.at[b], gsem.at[b])
        return carry

    lax.fori_loop(0, NCH // NBUF, body, 0)

    for b in range(NBUF):
        pltpu.make_async_copy(rows_v.at[b], out_hbm.at[wid, 0], ssem.at[b]).wait()


def kernel(category, weight):
    idx = category.reshape(NW, NCH, CH)
    out = _gather_kernel(idx, weight)
    return out.reshape(category.shape[0], category.shape[1], D)


# speedup vs baseline: 4.6664x; 1.0280x over previous
import functools

import jax
import jax.numpy as jnp
from jax import lax
from jax.experimental import pallas as pl
from jax.experimental.pallas import tpu as pltpu
from jax.experimental.pallas import tpu_sc as plsc

D = 64
B = 4096 * 50

_info = plsc.get_sparse_core_info()
_NC = _info.num_cores
_NS = _info.num_subcores
NW = _NC * _NS
BPW = B // NW
CH = 128
NCH = BPW // CH

NBUF = 5

_mesh = plsc.VectorSubcoreMesh(core_axis_name="c", subcore_axis_name="s")


@functools.partial(
    pl.kernel,
    mesh=_mesh,
    out_type=jax.ShapeDtypeStruct((NW, NCH, CH, D), jnp.float32),
    scratch_types=[
        pltpu.VMEM((NCH, CH), jnp.int32),
        pltpu.VMEM((NBUF, CH, D), jnp.float32),
        pltpu.SemaphoreType.DMA((NBUF,)),
        pltpu.SemaphoreType.DMA((NBUF,)),
    ],
    compiler_params=pltpu.CompilerParams(use_tc_tiling_on_sc=False),
)
def _gather_kernel(idx_hbm, table_hbm, out_hbm, idx_v, rows_v, gsem, ssem):
    wid = lax.axis_index("s") * _NC + lax.axis_index("c")
    pltpu.sync_copy(idx_hbm.at[wid], idx_v)

    for b in range(NBUF):
        pltpu.async_copy(table_hbm.at[idx_v.at[b]], rows_v.at[b], gsem.at[b])

    def body(k, carry):
        for b in range(NBUF):
            j = k * NBUF + b
            pltpu.make_async_copy(
                table_hbm.at[idx_v.at[b]], rows_v.at[b], gsem.at[b]).wait()
            pltpu.async_copy(rows_v.at[b], out_hbm.at[wid, j], ssem.at[b])

            @pl.when(j + NBUF < NCH)
            def _():
                pltpu.make_async_copy(
                    rows_v.at[b], out_hbm.at[wid, j], ssem.at[b]).wait()
                pltpu.async_copy(
                    table_hbm.at[idx_v.at[j + NBUF]], rows_v.at[b], gsem.at[b])
        return carry

    lax.fori_loop(0, NCH // NBUF, body, 0)

    for b in range(NBUF):
        pltpu.make_async_copy(rows_v.at[b], out_hbm.at[wid, 0], ssem.at[b]).wait()


def kernel(category, weight):
    idx = category.reshape(NW, NCH, CH)
    out = _gather_kernel(idx, weight)
    return out.reshape(category.shape[0], category.shape[1], D)


# speedup vs baseline: 4.6925x; 1.0056x over previous
import functools

import jax
import jax.numpy as jnp
from jax import lax
from jax.experimental import pallas as pl
from jax.experimental.pallas import tpu as pltpu
from jax.experimental.pallas import tpu_sc as plsc

D = 64
B = 4096 * 50

_info = plsc.get_sparse_core_info()
_NC = _info.num_cores
_NS = _info.num_subcores
NW = _NC * _NS
BPW = B // NW
CH = 256
NCH = BPW // CH

NBUF = 5

_mesh = plsc.VectorSubcoreMesh(core_axis_name="c", subcore_axis_name="s")


@functools.partial(
    pl.kernel,
    mesh=_mesh,
    out_type=jax.ShapeDtypeStruct((NW, NCH, CH, D), jnp.float32),
    scratch_types=[
        pltpu.VMEM((NCH, CH), jnp.int32),
        pltpu.VMEM((NBUF, CH, D), jnp.float32),
        pltpu.SemaphoreType.DMA((NBUF,)),
        pltpu.SemaphoreType.DMA((NBUF,)),
    ],
    compiler_params=pltpu.CompilerParams(use_tc_tiling_on_sc=False),
)
def _gather_kernel(idx_hbm, table_hbm, out_hbm, idx_v, rows_v, gsem, ssem):
    wid = lax.axis_index("s") * _NC + lax.axis_index("c")
    pltpu.sync_copy(idx_hbm.at[wid], idx_v)

    for b in range(NBUF):
        pltpu.async_copy(table_hbm.at[idx_v.at[b]], rows_v.at[b], gsem.at[b])

    def body(k, carry):
        for b in range(NBUF):
            j = k * NBUF + b
            pltpu.make_async_copy(
                table_hbm.at[idx_v.at[b]], rows_v.at[b], gsem.at[b]).wait()
            pltpu.async_copy(rows_v.at[b], out_hbm.at[wid, j], ssem.at[b])

            @pl.when(j + NBUF < NCH)
            def _():
                pltpu.make_async_copy(
                    rows_v.at[b], out_hbm.at[wid, j], ssem.at[b]).wait()
                pltpu.async_copy(
                    table_hbm.at[idx_v.at[j + NBUF]], rows_v.at[b], gsem.at[b])
        return carry

    lax.fori_loop(0, NCH // NBUF, body, 0)

    for b in range(NBUF):
        pltpu.make_async_copy(rows_v.at[b], out_hbm.at[wid, 0], ssem.at[b]).wait()


def kernel(category, weight):
    idx = category.reshape(NW, NCH, CH)
    out = _gather_kernel(idx, weight)
    return out.reshape(category.shape[0], category.shape[1], D)


# speedup vs baseline: 5.3177x; 1.1332x over previous
import functools

import jax
import jax.numpy as jnp
from jax import lax
from jax.experimental import pallas as pl
from jax.experimental.pallas import tpu as pltpu
from jax.experimental.pallas import tpu_sc as plsc

D = 64
NB = 4096
NS_ = 50
V = 100000

_info = plsc.get_sparse_core_info()
_NC = _info.num_cores
_NSUB = _info.num_subcores
NW = _NC * _NSUB
NPH = D // NW
NGRP = NB // 16

_mesh = plsc.VectorSubcoreMesh(core_axis_name="c", subcore_axis_name="s")


@functools.partial(
    pl.kernel,
    mesh=_mesh,
    out_type=jax.ShapeDtypeStruct((NS_, D, NB), jnp.float32),
    scratch_types=[
        pltpu.VMEM((V,), jnp.float32),
        pltpu.VMEM((2, NB), jnp.int32),
        pltpu.VMEM((2, NB), jnp.float32),
        pltpu.SemaphoreType.DMA,
        pltpu.SemaphoreType.DMA((2,)),
        pltpu.SemaphoreType.DMA((2,)),
    ],
    compiler_params=pltpu.CompilerParams(needs_layout_passes=False),
)
def _lookup_kernel(cat_hbm, tab_hbm, out_hbm, row_v, idx_v, res_v,
                   rsem, isem, osem):
    wid = lax.axis_index("s") * _NC + lax.axis_index("c")

    for p in range(NPH):
        d = wid + p * NW
        pltpu.async_copy(tab_hbm.at[d], row_v, rsem)
        for b in range(2):
            pltpu.async_copy(cat_hbm.at[b], idx_v.at[b], isem.at[b])
        pltpu.make_async_copy(tab_hbm.at[d], row_v, rsem).wait()

        def body(k, carry):
            for b in range(2):
                s = 2 * k + b
                pltpu.make_async_copy(
                    cat_hbm.at[s], idx_v.at[b], isem.at[b]).wait()

                @pl.when(k > 0)
                def _():
                    pltpu.make_async_copy(
                        res_v.at[b], out_hbm.at[s, d], osem.at[b]).wait()

                def grp(g, c):
                    off = pl.multiple_of(g * 16, 16)
                    idx = idx_v[b, pl.ds(off, 16)]
                    res_v[b, pl.ds(off, 16)] = plsc.load_gather(row_v, [idx])
                    return c

                lax.fori_loop(0, NGRP, grp, 0)
                pltpu.async_copy(res_v.at[b], out_hbm.at[s, d], osem.at[b])

                @pl.when(s + 2 < NS_)
                def _():
                    pltpu.async_copy(
                        cat_hbm.at[s + 2], idx_v.at[b], isem.at[b])
            return carry

        lax.fori_loop(0, NS_ // 2, body, 0)
        for b in range(2):
            pltpu.make_async_copy(
                res_v.at[b], out_hbm.at[0, d], osem.at[b]).wait()


def kernel(category, weight):
    out = _lookup_kernel(category.T, weight.T)
    return out.transpose(2, 0, 1)


# speedup vs baseline: 9.2719x; 1.7436x over previous
import functools

import jax
import jax.numpy as jnp
from jax import lax
from jax.experimental import pallas as pl
from jax.experimental.pallas import tpu as pltpu
from jax.experimental.pallas import tpu_sc as plsc

D = 64
NB = 4096
NS_ = 50
V = 100000

_info = plsc.get_sparse_core_info()
_NC = _info.num_cores
_NSUB = _info.num_subcores
NW = _NC * _NSUB
NPH = D // NW
NGRP = NB // 16

_mesh = plsc.VectorSubcoreMesh(core_axis_name="c", subcore_axis_name="s")


@functools.partial(
    pl.kernel,
    mesh=_mesh,
    out_type=jax.ShapeDtypeStruct((NS_, D, NB), jnp.float32),
    scratch_types=[
        pltpu.VMEM((V,), jnp.float32),
        pltpu.VMEM((2, NB), jnp.int32),
        pltpu.VMEM((2, NB), jnp.float32),
        pltpu.SemaphoreType.DMA,
        pltpu.SemaphoreType.DMA((2,)),
        pltpu.SemaphoreType.DMA((2,)),
    ],
    compiler_params=pltpu.CompilerParams(needs_layout_passes=False),
)
def _lookup_kernel(cat_hbm, tab_hbm, out_hbm, row_v, idx_v, res_v,
                   rsem, isem, osem):
    wid = lax.axis_index("s") * _NC + lax.axis_index("c")

    for p in range(NPH):
        d = wid + p * NW
        pltpu.async_copy(tab_hbm.at[d], row_v, rsem)
        for b in range(2):
            pltpu.async_copy(cat_hbm.at[b], idx_v.at[b], isem.at[b])
        pltpu.make_async_copy(tab_hbm.at[d], row_v, rsem).wait()

        def body(k, carry):
            for b in range(2):
                s = 2 * k + b
                pltpu.make_async_copy(
                    cat_hbm.at[s], idx_v.at[b], isem.at[b]).wait()

                @pl.when(k > 0)
                def _():
                    pltpu.make_async_copy(
                        res_v.at[b], out_hbm.at[s, d], osem.at[b]).wait()

                @plsc.parallel_loop(0, NGRP, unroll=16)
                def grp(g):
                    off = pl.multiple_of(g * 16, 16)
                    idx = idx_v[b, pl.ds(off, 16)]
                    res_v[b, pl.ds(off, 16)] = plsc.load_gather(row_v, [idx])
                pltpu.async_copy(res_v.at[b], out_hbm.at[s, d], osem.at[b])

                @pl.when(s + 2 < NS_)
                def _():
                    pltpu.async_copy(
                        cat_hbm.at[s + 2], idx_v.at[b], isem.at[b])
            return carry

        lax.fori_loop(0, NS_ // 2, body, 0)
        for b in range(2):
            pltpu.make_async_copy(
                res_v.at[b], out_hbm.at[0, d], osem.at[b]).wait()


def kernel(category, weight):
    out = _lookup_kernel(category.T, weight.T)
    return out.transpose(2, 0, 1)
